# Initial kernel scaffold; baseline (speedup 1.0000x reference)
import jax
import jax.numpy as jnp
from jax.experimental import pallas as pl


def kernel(x, id_table, pos_table, ln_gamma, ln_beta):
    raise NotImplementedError("write your pallas kernel here")


# speedup vs baseline: 1.7743x; 1.7743x over previous
import functools

import jax
import jax.numpy as jnp
from jax import lax
from jax.experimental import pallas as pl
from jax.experimental.pallas import tpu as pltpu
from jax.experimental.pallas import tpu_sc as plsc

EMBED = 128
SEQ = 200
LANES = 16
NV = EMBED // LANES
EPS = 1e-12


_GATHER_DNUMS = lax.GatherDimensionNumbers(
    offset_dims=(), collapsed_slice_dims=(0,), start_index_map=(0,))


def _lane_perm(v, idx):
    return lax.gather(
        v, idx[:, None], dimension_numbers=_GATHER_DNUMS, slice_sizes=(1,),
        mode=lax.GatherScatterMode.PROMISE_IN_BOUNDS)


def _lane_sum(v):
    for k in (1, 2, 4, 8):
        idx = jnp.arange(LANES, dtype=jnp.int32) ^ k
        v = v + _lane_perm(v, idx)
    return v


def _rsqrt(v):
    i = lax.bitcast_convert_type(v, jnp.int32)
    i = jnp.int32(0x5F3759DF) - lax.shift_right_arithmetic(i, 1)
    y = lax.bitcast_convert_type(i, jnp.float32)
    for _ in range(3):
        y = y * (1.5 - 0.5 * v * y * y)
    return y


@functools.lru_cache(maxsize=None)
def _make_kernel(n_tokens):
    info = plsc.get_sparse_core_info()
    n_workers = info.num_cores * info.num_subcores
    tokens_per_w = n_tokens // n_workers
    seqs_per_w = tokens_per_w // SEQ
    mesh = plsc.VectorSubcoreMesh(core_axis_name="c", subcore_axis_name="s")

    @functools.partial(
        pl.kernel,
        mesh=mesh,
        out_type=jax.ShapeDtypeStruct((n_tokens, EMBED), jnp.float32),
        scratch_types=[
            pltpu.VMEM((SEQ,), jnp.int32),
            pltpu.VMEM((SEQ, EMBED), jnp.float32),
            pltpu.VMEM((SEQ, EMBED), jnp.float32),
            pltpu.VMEM((EMBED,), jnp.float32),
            pltpu.VMEM((EMBED,), jnp.float32),
            pltpu.SemaphoreType.DMA,
        ],
    )
    def k(x_hbm, tab_hbm, pos_hbm, g_hbm, b_hbm, out_hbm,
          idx_v, rows_v, pos_v, g_v, b_v, sem):
        wid = lax.axis_index("s") * info.num_cores + lax.axis_index("c")
        pltpu.sync_copy(pos_hbm, pos_v)
        pltpu.sync_copy(g_hbm, g_v)
        pltpu.sync_copy(b_hbm, b_v)
        gamma = [g_v[pl.ds(LANES * j, LANES)] for j in range(NV)]
        beta = [b_v[pl.ds(LANES * j, LANES)] for j in range(NV)]

        def seq_body(i, carry):
            base = (wid * seqs_per_w + i) * SEQ
            pltpu.sync_copy(x_hbm.at[pl.ds(base, SEQ)], idx_v)
            c1 = pltpu.async_copy(tab_hbm.at[idx_v.at[pl.ds(0, 128)]],
                                  rows_v.at[pl.ds(0, 128)], sem)
            c2 = pltpu.async_copy(tab_hbm.at[idx_v.at[pl.ds(128, SEQ - 128)]],
                                  rows_v.at[pl.ds(128, SEQ - 128)], sem)
            c1.wait()
            c2.wait()

            def tok_body(t, carry2):
                v = [rows_v[t, pl.ds(LANES * j, LANES)]
                     + pos_v[t, pl.ds(LANES * j, LANES)]
                     for j in range(NV)]
                s = v[0]
                for j in range(1, NV):
                    s = s + v[j]
                mean = _lane_sum(s) * (1.0 / EMBED)
                d = [vj - mean for vj in v]
                sq = d[0] * d[0]
                for j in range(1, NV):
                    sq = sq + d[j] * d[j]
                var = _lane_sum(sq) * (1.0 / EMBED)
                rstd = _rsqrt(var + EPS)
                for j in range(NV):
                    rows_v[t, pl.ds(LANES * j, LANES)] = (
                        d[j] * rstd * gamma[j] + beta[j])
                return carry2

            lax.fori_loop(0, SEQ, tok_body, 0)
            pltpu.sync_copy(rows_v, out_hbm.at[pl.ds(base, SEQ)])
            return carry

        lax.fori_loop(0, seqs_per_w, seq_body, 0)

    return k


def kernel(x, id_table, pos_table, ln_gamma, ln_beta):
    batch, seq_len = x.shape
    n_tokens = batch * seq_len
    out = _make_kernel(n_tokens)(
        x.reshape(-1), id_table, pos_table, ln_gamma, ln_beta)
    return out.reshape(batch, seq_len, EMBED)


# speedup vs baseline: 5.0407x; 2.8409x over previous
import functools

import jax
import jax.numpy as jnp
from jax import lax
from jax.experimental import pallas as pl
from jax.experimental.pallas import tpu as pltpu
from jax.experimental.pallas import tpu_sc as plsc

EMBED = 128
SEQ = 200
LANES = 16
NV = EMBED // LANES
EPS = 1e-12
NBUF = 3


_GATHER_DNUMS = lax.GatherDimensionNumbers(
    offset_dims=(), collapsed_slice_dims=(0,), start_index_map=(0,))


def _lane_perm(v, idx):
    return lax.gather(
        v, idx[:, None], dimension_numbers=_GATHER_DNUMS, slice_sizes=(1,),
        mode=lax.GatherScatterMode.PROMISE_IN_BOUNDS)


def _lane_sum(v):
    for k in (1, 2, 4, 8):
        idx = jnp.arange(LANES, dtype=jnp.int32) ^ k
        v = v + _lane_perm(v, idx)
    return v


def _rsqrt(v):
    i = lax.bitcast_convert_type(v, jnp.int32)
    i = jnp.int32(0x5F3759DF) - lax.shift_right_arithmetic(i, 1)
    y = lax.bitcast_convert_type(i, jnp.float32)
    for _ in range(2):
        y = y * (1.5 - 0.5 * v * y * y)
    return y


def _tree_sum(vs):
    vs = list(vs)
    while len(vs) > 1:
        vs = [vs[i] + vs[i + 1] for i in range(0, len(vs) - 1, 2)] + (
            [vs[-1]] if len(vs) % 2 else [])
    return vs[0]


@functools.lru_cache(maxsize=None)
def _make_kernel(n_tokens):
    info = plsc.get_sparse_core_info()
    n_workers = info.num_cores * info.num_subcores
    tokens_per_w = n_tokens // n_workers
    seqs_per_w = tokens_per_w // SEQ
    mesh = plsc.VectorSubcoreMesh(core_axis_name="c", subcore_axis_name="s")

    @functools.partial(
        pl.kernel,
        mesh=mesh,
        out_type=jax.ShapeDtypeStruct((n_tokens, EMBED), jnp.float32),
        scratch_types=(
            [pltpu.VMEM((tokens_per_w,), jnp.int32)]
            + [pltpu.VMEM((SEQ, EMBED), jnp.float32) for _ in range(NBUF)]
            + [pltpu.VMEM((SEQ, EMBED), jnp.float32),
               pltpu.VMEM((EMBED,), jnp.float32),
               pltpu.VMEM((EMBED,), jnp.float32)]
            + [pltpu.SemaphoreType.DMA for _ in range(2 * NBUF)]
        ),
    )
    def k(x_hbm, tab_hbm, pos_hbm, g_hbm, b_hbm, out_hbm,
          idx_all, r0, r1, r2, pos_v, g_v, b_v,
          gs0, gs1, gs2, ws0, ws1, ws2):
        rows = [r0, r1, r2]
        gsem = [gs0, gs1, gs2]
        wsem = [ws0, ws1, ws2]
        wid = lax.axis_index("s") * info.num_cores + lax.axis_index("c")
        tok0 = wid * tokens_per_w
        pltpu.sync_copy(x_hbm.at[pl.ds(tok0, tokens_per_w)], idx_all)
        pltpu.sync_copy(pos_hbm, pos_v)
        pltpu.sync_copy(g_hbm, g_v)
        pltpu.sync_copy(b_hbm, b_v)
        gamma = [g_v[pl.ds(LANES * j, LANES)] for j in range(NV)]
        beta = [b_v[pl.ds(LANES * j, LANES)] for j in range(NV)]

        def issue_g(i, b):
            off = i * SEQ
            pltpu.async_copy(tab_hbm.at[idx_all.at[pl.ds(off, 128)]],
                             rows[b].at[pl.ds(0, 128)], gsem[b])
            pltpu.async_copy(tab_hbm.at[idx_all.at[pl.ds(off + 128, SEQ - 128)]],
                             rows[b].at[pl.ds(128, SEQ - 128)], gsem[b])

        def wait_g(b):
            pltpu.make_async_copy(tab_hbm.at[pl.ds(0, SEQ)], rows[b],
                                  gsem[b]).wait()

        def issue_w(i, b):
            pltpu.async_copy(rows[b], out_hbm.at[pl.ds(tok0 + i * SEQ, SEQ)],
                             wsem[b])

        def wait_w(b):
            pltpu.make_async_copy(rows[b], out_hbm.at[pl.ds(0, SEQ)],
                                  wsem[b]).wait()

        def compute(i, b):
            rv = rows[b]

            @plsc.parallel_loop(0, SEQ, unroll=4)
            def tok_body(t):
                v = [rv[t, pl.ds(LANES * j, LANES)]
                     + pos_v[t, pl.ds(LANES * j, LANES)]
                     for j in range(NV)]
                s = _lane_sum(_tree_sum(v))
                q = _lane_sum(_tree_sum([vj * vj for vj in v]))
                mean = s * (1.0 / EMBED)
                var = q * (1.0 / EMBED) - mean * mean + EPS
                rstd = _rsqrt(var)
                a = [rstd * gamma[j] for j in range(NV)]
                for j in range(NV):
                    rv[t, pl.ds(LANES * j, LANES)] = (
                        (v[j] - mean) * a[j] + beta[j])

        issue_g(0, 0)
        issue_g(1, 1)
        issue_g(2, 2)
        wait_g(0)
        compute(0, 0)
        issue_w(0, 0)
        for i in (1, 2):
            b = i % NBUF
            nb = (i + 2) % NBUF
            wait_w(nb)
            issue_g(i + 2, nb)
            wait_g(b)
            compute(i, b)
            issue_w(i, b)

        def steady(g, carry):
            for kk in range(NBUF):
                i = 3 + 3 * g + kk
                nb = (kk + 2) % NBUF
                wait_w(nb)
                issue_g(i + 2, nb)
                wait_g(kk)
                compute(i, kk)
                issue_w(i, kk)
            return carry

        n_steady = (seqs_per_w - 5) // NBUF
        lax.fori_loop(0, n_steady, steady, 0)

        for i in (seqs_per_w - 2, seqs_per_w - 1):
            b = i % NBUF
            wait_g(b)
            compute(i, b)
            issue_w(i, b)
        for b in range(NBUF):
            wait_w(b)

    return k


def kernel(x, id_table, pos_table, ln_gamma, ln_beta):
    batch, seq_len = x.shape
    n_tokens = batch * seq_len
    out = _make_kernel(n_tokens)(
        x.reshape(-1), id_table, pos_table, ln_gamma, ln_beta)
    return out.reshape(batch, seq_len, EMBED)
